# Initial kernel scaffold; baseline (speedup 1.0000x reference)
import jax
import jax.numpy as jnp
from jax.experimental import pallas as pl


def kernel(inputs):
    raise NotImplementedError("write your pallas kernel here")


# speedup vs baseline: 79.6151x; 79.6151x over previous
import functools

import jax
import jax.numpy as jnp
from jax import lax
from jax.experimental import pallas as pl
from jax.experimental.pallas import tpu as pltpu
from jax.experimental.pallas import tpu_sc as plsc

H = 64
W = 64
BIMG = 8
BIG = 2 ** 30


def _shift(x, d, axis, fill, backward):
    n = x.shape[axis]
    pad_shape = list(x.shape)
    pad_shape[axis] = d
    pad = jnp.full(pad_shape, fill, x.dtype)
    if backward:
        return jnp.concatenate([lax.slice_in_dim(x, d, n, axis=axis), pad], axis=axis)
    return jnp.concatenate([pad, lax.slice_in_dim(x, 0, n - d, axis=axis)], axis=axis)


def _seg_scan_min(v0, f0, axis, backward):
    v, f = v0, f0
    d = 1
    n = v0.shape[axis]
    while d < n:
        vs = _shift(v, d, axis, BIG, backward)
        fs = _shift(f, d, axis, 1, backward)
        v = jnp.where(f > 0, v, jnp.minimum(v, vs))
        f = jnp.maximum(f, fs)
        d *= 2
    return v


def _run_min_pass(l, notfg, axis):
    fwd = _seg_scan_min(l, notfg, axis, backward=False)
    bwd = _seg_scan_min(l, notfg, axis, backward=True)
    return jnp.minimum(fwd, bwd)


def _cumsum(x, axis):
    n = x.shape[axis]
    d = 1
    while d < n:
        x = x + _shift(x, d, axis, jnp.int32(0), backward=False)
        d *= 2
    return x


def _label_block_kernel(fg_ref, lab_ref, rank_ref, base_ref):
    pid = pl.program_id(0)

    @pl.when(pid == 0)
    def _():
        base_ref[0] = jnp.int32(0)

    fg = fg_ref[...] != 0
    notfg = 1 - fg_ref[...].astype(jnp.int32)

    img = lax.broadcasted_iota(jnp.int32, (BIMG, H, W), 0)
    row = lax.broadcasted_iota(jnp.int32, (BIMG, H, W), 1)
    col = lax.broadcasted_iota(jnp.int32, (BIMG, H, W), 2)
    gidx = (pid * BIMG + img) * (H * W) + row * W + col
    lab0 = jnp.where(fg, gidx + 1, BIG)

    def one_iter(l):
        l = _run_min_pass(l, notfg, axis=2)
        l = _run_min_pass(l, notfg, axis=1)
        return l

    def cond_fn(carry):
        prev, cur = carry
        return jnp.any(prev != cur)

    def body_fn(carry):
        _, cur = carry
        return cur, one_iter(cur)

    _, lab = lax.while_loop(cond_fn, body_fn, (lab0, one_iter(lab0)))

    is_root = jnp.logical_and(fg, lab == gidx + 1)
    r01 = is_root.astype(jnp.int32)

    inrow = _cumsum(r01, axis=2)
    rowtot = inrow[:, :, W - 1]
    rows_inc = _cumsum(rowtot, axis=1)
    imgtot = rows_inc[:, H - 1]
    imgs_inc = _cumsum(imgtot, axis=0)
    img_excl = imgs_inc - imgtot
    rows_excl = rows_inc - rowtot + img_excl[:, None]

    base = base_ref[0]
    rank = base + rows_excl[:, :, None] + inrow
    block_total = imgs_inc[BIMG - 1]
    base_ref[0] = base + block_total

    lab_ref[...] = jnp.where(fg, lab, 0)
    rank_ref[...] = rank


def _tc_label(fg):
    n = fg.shape[0]
    grid = n // BIMG
    return pl.pallas_call(
        _label_block_kernel,
        grid=(grid,),
        in_specs=[pl.BlockSpec((BIMG, H, W), lambda i: (i, 0, 0))],
        out_specs=[
            pl.BlockSpec((BIMG, H, W), lambda i: (i, 0, 0)),
            pl.BlockSpec((BIMG, H, W), lambda i: (i, 0, 0)),
        ],
        out_shape=[
            jax.ShapeDtypeStruct((n, H, W), jnp.int32),
            jax.ShapeDtypeStruct((n, H, W), jnp.int32),
        ],
        scratch_shapes=[pltpu.SMEM((1,), jnp.int32)],
    )(fg)


def _sc_relabel(lab, rank):
    n = lab.shape[0]
    hw = H * W
    info = plsc.get_sparse_core_info()
    nworkers = info.num_cores * info.num_subcores
    per_w = n // nworkers
    mesh = plsc.VectorSubcoreMesh(core_axis_name="c", subcore_axis_name="s")

    @functools.partial(
        pl.kernel,
        mesh=mesh,
        compiler_params=pltpu.CompilerParams(needs_layout_passes=False),
        out_type=jax.ShapeDtypeStruct((n, hw), jnp.int32),
        scratch_types=[
            pltpu.VMEM((hw,), jnp.int32),
            pltpu.VMEM((hw,), jnp.int32),
            pltpu.VMEM((hw,), jnp.int32),
        ],
    )
    def k(lab_hbm, rank_hbm, out_hbm, lab_v, rank_v, out_v):
        wid = lax.axis_index("s") * info.num_cores + lax.axis_index("c")

        def per_image(i, _):
            img = wid * per_w + i
            pltpu.sync_copy(lab_hbm.at[img], lab_v)
            pltpu.sync_copy(rank_hbm.at[img], rank_v)
            img_base = img * hw + 1

            def per_vec(j, _):
                l = lab_v[pl.ds(j * 16, 16)]
                idx = jnp.maximum(l - img_base, 0)
                g = plsc.load_gather(rank_v, [idx])
                out_v[pl.ds(j * 16, 16)] = jnp.where(l > 0, g, 0)
                return 0

            lax.fori_loop(0, hw // 16, per_vec, 0, unroll=4)
            pltpu.sync_copy(out_v, out_hbm.at[img])
            return 0

        lax.fori_loop(0, per_w, per_image, 0)

    return k(lab, rank)


def kernel(inputs):
    b, sn, s1, s2 = inputs.shape
    n = b * sn
    fg = inputs.reshape(n, s1, s2).astype(jnp.uint8)
    lab, rank = _tc_label(fg)
    out = _sc_relabel(lab.reshape(n, H * W), rank.reshape(n, H * W))
    return out.reshape(b, sn, s1, s2)


# speedup vs baseline: 114.3797x; 1.4367x over previous
import functools

import jax
import jax.numpy as jnp
from jax import lax
from jax.experimental import pallas as pl
from jax.experimental.pallas import tpu as pltpu
from jax.experimental.pallas import tpu_sc as plsc

H = 64
W = 64
HW = H * W
PW = 2 * W
BPAIR = 4
BIG = 2 ** 30


def _shift(x, d, axis, fill, backward):
    n = x.shape[axis]
    pad_shape = list(x.shape)
    pad_shape[axis] = d
    pad = jnp.full(pad_shape, fill, x.dtype)
    if backward:
        return jnp.concatenate([lax.slice_in_dim(x, d, n, axis=axis), pad], axis=axis)
    return jnp.concatenate([pad, lax.slice_in_dim(x, 0, n - d, axis=axis)], axis=axis)


def _flag_pyramid(f0big, axis, backward, dists):
    out = [f0big]
    f = f0big
    for d in dists[:-1]:
        f = jnp.maximum(f, _shift(f, d, axis, BIG, backward))
        out.append(f)
    return out


def _seg_scan_min(v, fbs, axis, backward, dists):
    for d, fb in zip(dists, fbs):
        v = jnp.minimum(v, jnp.maximum(_shift(v, d, axis, BIG, backward), fb))
    return v


def _cumsum(x, axis):
    n = x.shape[axis]
    d = 1
    while d < n:
        x = x + _shift(x, d, axis, 0, backward=False)
        d *= 2
    return x


_ROW_D = (1, 2, 4, 8, 16, 32)
_COL_D = (1, 2, 4, 8, 16, 32)


def _label_block_kernel(fg_ref, lab_ref, rank_ref, base_ref):
    pid = pl.program_id(0)

    @pl.when(pid == 0)
    def _():
        base_ref[0] = jnp.int32(0)

    shape = (BPAIR, H, PW)
    fgi = fg_ref[...].astype(jnp.int32)
    fg = fgi != 0
    notfg_big = (1 - fgi) * BIG

    b = lax.broadcasted_iota(jnp.int32, shape, 0)
    row = lax.broadcasted_iota(jnp.int32, shape, 1)
    col = lax.broadcasted_iota(jnp.int32, shape, 2)
    half = (col >= W).astype(jnp.int32)
    img = (pid * BPAIR + b) * 2 + half
    gidx = img * HW + row * W + (col & (W - 1))
    lab0 = jnp.where(fg, gidx + 1, BIG)

    rf0 = jnp.maximum(notfg_big, jnp.where(col == W, BIG, 0))
    rb0 = jnp.maximum(notfg_big, jnp.where(col == W - 1, BIG, 0))
    fb_rf = _flag_pyramid(rf0, 2, False, _ROW_D)
    fb_rb = _flag_pyramid(rb0, 2, True, _ROW_D)
    fb_cf = _flag_pyramid(notfg_big, 1, False, _COL_D)
    fb_cb = _flag_pyramid(notfg_big, 1, True, _COL_D)

    def one_iter(l):
        fwd = _seg_scan_min(l, fb_rf, 2, False, _ROW_D)
        bwd = _seg_scan_min(l, fb_rb, 2, True, _ROW_D)
        l = jnp.minimum(fwd, bwd)
        fwd = _seg_scan_min(l, fb_cf, 1, False, _COL_D)
        bwd = _seg_scan_min(l, fb_cb, 1, True, _COL_D)
        return jnp.minimum(fwd, bwd)

    def cond_fn(carry):
        prev, cur = carry
        return jnp.any(prev != cur)

    def body_fn(carry):
        _, cur = carry
        return cur, one_iter(cur)

    _, lab = lax.while_loop(cond_fn, body_fn, (lab0, one_iter(lab0)))

    r01 = jnp.where(jnp.logical_and(fg, lab == gidx + 1), 1, 0)

    inrow = _cumsum(r01, axis=2)
    ra = inrow[:, :, W - 1]
    rfull = inrow[:, :, PW - 1]
    rb = rfull - ra
    rows_inc_a = _cumsum(ra, axis=1)
    rows_inc_b = _cumsum(rb, axis=1)
    tot_a = rows_inc_a[:, H - 1]
    tot_b = rows_inc_b[:, H - 1]
    pair_tot = tot_a + tot_b
    pairs_inc = _cumsum(pair_tot, axis=0)
    pair_excl = pairs_inc - pair_tot
    add_a = rows_inc_a - ra + pair_excl[:, None]
    add_b = rows_inc_b - rb + pair_excl[:, None] + tot_a[:, None] - ra

    base = base_ref[0]
    rank = base + inrow + jnp.where(col < W, add_a[:, :, None], add_b[:, :, None])
    base_ref[0] = base + pairs_inc[BPAIR - 1]

    lab_ref[...] = jnp.where(fg, lab, 0)
    rank_ref[...] = rank


def _tc_label(fgp):
    np_ = fgp.shape[0]
    grid = np_ // BPAIR
    return pl.pallas_call(
        _label_block_kernel,
        grid=(grid,),
        in_specs=[pl.BlockSpec((BPAIR, H, PW), lambda i: (i, 0, 0))],
        out_specs=[
            pl.BlockSpec((BPAIR, H, PW), lambda i: (i, 0, 0)),
            pl.BlockSpec((BPAIR, H, PW), lambda i: (i, 0, 0)),
        ],
        out_shape=[
            jax.ShapeDtypeStruct((np_, H, PW), jnp.int32),
            jax.ShapeDtypeStruct((np_, H, PW), jnp.int32),
        ],
        scratch_shapes=[pltpu.SMEM((1,), jnp.int32)],
    )(fgp)


def _sc_relabel(lab, rank, n_img):
    npair = lab.shape[0]
    phw = H * PW
    info = plsc.get_sparse_core_info()
    nworkers = info.num_cores * info.num_subcores
    per_w = npair // nworkers
    mesh = plsc.VectorSubcoreMesh(core_axis_name="c", subcore_axis_name="s")

    @functools.partial(
        pl.kernel,
        mesh=mesh,
        compiler_params=pltpu.CompilerParams(needs_layout_passes=False),
        out_type=jax.ShapeDtypeStruct((n_img, HW), jnp.int32),
        scratch_types=[
            pltpu.VMEM((phw,), jnp.int32),
            pltpu.VMEM((phw,), jnp.int32),
            pltpu.VMEM((HW,), jnp.int32),
        ],
    )
    def k(lab_hbm, rank_hbm, out_hbm, lab_v, rank_v, out_v):
        wid = lax.axis_index("s") * info.num_cores + lax.axis_index("c")

        def per_pair(i, _):
            p = wid * per_w + i
            pltpu.sync_copy(lab_hbm.at[p], lab_v)
            pltpu.sync_copy(rank_hbm.at[p], rank_v)

            def per_half(h, _):
                img_base = (2 * p + h) * HW + 1
                hoff = h * W

                def per_vec(q, _):
                    off = (q >> 2) * PW + hoff + (q & 3) * 16
                    l = lab_v[pl.ds(off, 16)]
                    idx = jnp.maximum(l - img_base, 0)
                    idx_p = ((idx >> 6) << 7) + (hoff + (idx & (W - 1)))
                    g = plsc.load_gather(rank_v, [idx_p])
                    out_v[pl.ds(q * 16, 16)] = jnp.where(l > 0, g, 0)
                    return 0

                lax.fori_loop(0, HW // 16, per_vec, 0, unroll=4)
                pltpu.sync_copy(out_v, out_hbm.at[2 * p + h])
                return 0

            lax.fori_loop(0, 2, per_half, 0)
            return 0

        lax.fori_loop(0, per_w, per_pair, 0)

    return k(lab, rank)


def kernel(inputs):
    b, sn, s1, s2 = inputs.shape
    n = b * sn
    fg = inputs.reshape(n // 2, 2, s1, s2).astype(jnp.uint8)
    fgp = fg.transpose(0, 2, 1, 3).reshape(n // 2, H, PW)
    lab, rank = _tc_label(fgp)
    out = _sc_relabel(lab.reshape(n // 2, H * PW), rank.reshape(n // 2, H * PW), n)
    return out.reshape(b, sn, s1, s2)


# speedup vs baseline: 120.9942x; 1.0578x over previous
import functools

import jax
import jax.numpy as jnp
from jax import lax
from jax.experimental import pallas as pl
from jax.experimental.pallas import tpu as pltpu
from jax.experimental.pallas import tpu_sc as plsc

H = 64
W = 64
HW = H * W
PW = 2 * W
BPAIR = 4
BIG = 2 ** 31 - 1


def _roll(x, d, axis, backward):
    n = x.shape[axis]
    return pltpu.roll(x, (n - d) if backward else d, axis)


def _flag_pyramid(f0big, axis, backward, dists):
    out = [f0big]
    f = f0big
    for d in dists[:-1]:
        f = jnp.bitwise_or(f, _roll(f, d, axis, backward))
        out.append(f)
    return out


def _seg_scan_min(v, fbs, axis, backward, dists):
    for d, fb in zip(dists, fbs):
        v = jnp.minimum(v, jnp.bitwise_or(_roll(v, d, axis, backward), fb))
    return v


def _shift(x, d, axis, fill, backward):
    n = x.shape[axis]
    pad_shape = list(x.shape)
    pad_shape[axis] = d
    pad = jnp.full(pad_shape, fill, x.dtype)
    if backward:
        return jnp.concatenate([lax.slice_in_dim(x, d, n, axis=axis), pad], axis=axis)
    return jnp.concatenate([pad, lax.slice_in_dim(x, 0, n - d, axis=axis)], axis=axis)


def _cumsum(x, axis):
    n = x.shape[axis]
    d = 1
    while d < n:
        x = x + _shift(x, d, axis, 0, backward=False)
        d *= 2
    return x


_ROW_D = (1, 2, 4, 8, 16, 32)
_COL_D = (1, 2, 4, 8, 16, 32)


def _label_block_kernel(fg_ref, lab_ref, rank_ref, base_ref):
    pid = pl.program_id(0)

    @pl.when(pid == 0)
    def _():
        base_ref[0] = jnp.int32(0)

    shape = (BPAIR, H, PW)
    fgi = fg_ref[...].astype(jnp.int32)
    fg = fgi != 0
    notfg_big = (1 - fgi) * BIG

    b = lax.broadcasted_iota(jnp.int32, shape, 0)
    row = lax.broadcasted_iota(jnp.int32, shape, 1)
    col = lax.broadcasted_iota(jnp.int32, shape, 2)
    half = (col >= W).astype(jnp.int32)
    img = (pid * BPAIR + b) * 2 + half
    gidx = img * HW + row * W + (col & (W - 1))
    lab0 = jnp.where(fg, gidx + 1, BIG)

    rf0 = jnp.bitwise_or(notfg_big, jnp.where((col == W) | (col == 0), BIG, 0))
    rb0 = jnp.bitwise_or(notfg_big,
                         jnp.where((col == W - 1) | (col == PW - 1), BIG, 0))
    cf0 = jnp.bitwise_or(notfg_big, jnp.where(row == 0, BIG, 0))
    cb0 = jnp.bitwise_or(notfg_big, jnp.where(row == H - 1, BIG, 0))
    fb_rf = _flag_pyramid(rf0, 2, False, _ROW_D)
    fb_rb = _flag_pyramid(rb0, 2, True, _ROW_D)
    fb_cf = _flag_pyramid(cf0, 1, False, _COL_D)
    fb_cb = _flag_pyramid(cb0, 1, True, _COL_D)

    def one_iter(l):
        fwd = _seg_scan_min(l, fb_rf, 2, False, _ROW_D)
        bwd = _seg_scan_min(l, fb_rb, 2, True, _ROW_D)
        l = jnp.minimum(fwd, bwd)
        fwd = _seg_scan_min(l, fb_cf, 1, False, _COL_D)
        bwd = _seg_scan_min(l, fb_cb, 1, True, _COL_D)
        return jnp.minimum(fwd, bwd)

    def cond_fn(carry):
        prev, cur = carry
        return jnp.any(prev != cur)

    def body_fn(carry):
        _, cur = carry
        return cur, one_iter(cur)

    _, lab = lax.while_loop(cond_fn, body_fn, (lab0, one_iter(lab0)))

    r01 = jnp.where(jnp.logical_and(fg, lab == gidx + 1), 1, 0)

    inrow = _cumsum(r01, axis=2)
    ra = inrow[:, :, W - 1]
    rfull = inrow[:, :, PW - 1]
    rb = rfull - ra
    rows_inc_a = _cumsum(ra, axis=1)
    rows_inc_b = _cumsum(rb, axis=1)
    tot_a = rows_inc_a[:, H - 1]
    tot_b = rows_inc_b[:, H - 1]
    pair_tot = tot_a + tot_b
    pairs_inc = _cumsum(pair_tot, axis=0)
    pair_excl = pairs_inc - pair_tot
    add_a = rows_inc_a - ra + pair_excl[:, None]
    add_b = rows_inc_b - rb + pair_excl[:, None] + tot_a[:, None] - ra

    base = base_ref[0]
    rank = base + inrow + jnp.where(col < W, add_a[:, :, None], add_b[:, :, None])
    base_ref[0] = base + pairs_inc[BPAIR - 1]

    lab_ref[...] = jnp.where(fg, lab, 0)
    rank_ref[...] = rank


def _tc_label(fgp):
    np_ = fgp.shape[0]
    grid = np_ // BPAIR
    return pl.pallas_call(
        _label_block_kernel,
        grid=(grid,),
        in_specs=[pl.BlockSpec((BPAIR, H, PW), lambda i: (i, 0, 0))],
        out_specs=[
            pl.BlockSpec((BPAIR, H, PW), lambda i: (i, 0, 0)),
            pl.BlockSpec((BPAIR, H, PW), lambda i: (i, 0, 0)),
        ],
        out_shape=[
            jax.ShapeDtypeStruct((np_, H, PW), jnp.int32),
            jax.ShapeDtypeStruct((np_, H, PW), jnp.int32),
        ],
        scratch_shapes=[pltpu.SMEM((1,), jnp.int32)],
    )(fgp)


def _sc_relabel(lab, rank, n_img):
    npair = lab.shape[0]
    phw = H * PW
    info = plsc.get_sparse_core_info()
    nworkers = info.num_cores * info.num_subcores
    per_w = npair // nworkers
    mesh = plsc.VectorSubcoreMesh(core_axis_name="c", subcore_axis_name="s")

    @functools.partial(
        pl.kernel,
        mesh=mesh,
        compiler_params=pltpu.CompilerParams(needs_layout_passes=False),
        out_type=jax.ShapeDtypeStruct((n_img, HW), jnp.int32),
        scratch_types=[
            pltpu.VMEM((phw,), jnp.int32),
            pltpu.VMEM((phw,), jnp.int32),
            pltpu.VMEM((HW,), jnp.int32),
        ],
    )
    def k(lab_hbm, rank_hbm, out_hbm, lab_v, rank_v, out_v):
        wid = lax.axis_index("s") * info.num_cores + lax.axis_index("c")

        def per_pair(i, _):
            p = wid * per_w + i
            pltpu.sync_copy(lab_hbm.at[p], lab_v)
            pltpu.sync_copy(rank_hbm.at[p], rank_v)

            def per_half(h, _):
                img_base = (2 * p + h) * HW + 1
                hoff = h * W

                def per_vec(q, _):
                    off = (q >> 2) * PW + hoff + (q & 3) * 16
                    l = lab_v[pl.ds(off, 16)]
                    idx = jnp.maximum(l - img_base, 0)
                    idx_p = ((idx >> 6) << 7) + (hoff + (idx & (W - 1)))
                    g = plsc.load_gather(rank_v, [idx_p])
                    out_v[pl.ds(q * 16, 16)] = jnp.where(l > 0, g, 0)
                    return 0

                lax.fori_loop(0, HW // 16, per_vec, 0, unroll=4)
                pltpu.sync_copy(out_v, out_hbm.at[2 * p + h])
                return 0

            lax.fori_loop(0, 2, per_half, 0)
            return 0

        lax.fori_loop(0, per_w, per_pair, 0)

    return k(lab, rank)


def kernel(inputs):
    b, sn, s1, s2 = inputs.shape
    n = b * sn
    fg = inputs.reshape(n // 2, 2, s1, s2).astype(jnp.uint8)
    fgp = fg.transpose(0, 2, 1, 3).reshape(n // 2, H, PW)
    lab, rank = _tc_label(fgp)
    out = _sc_relabel(lab.reshape(n // 2, H * PW), rank.reshape(n // 2, H * PW), n)
    return out.reshape(b, sn, s1, s2)
